# baseline (speedup 1.0000x reference)
import jax
import jax.numpy as jnp
from jax.experimental import pallas as pl
from jax.experimental.pallas import tpu as pltpu

DIM = 2048
N_EXPERTS = 64
TOPK = 2
TOKENS = 16384

BLOCK_T = 1024


def _router_kernel(x_ref, w_ref, out_w_ref, out_i_ref):
    x = x_ref[...]
    w = w_ref[...]
    logits = jax.lax.dot_general(
        x, w,
        dimension_numbers=(((1,), (1,)), ((), ())),
        preferred_element_type=jnp.float32,
    )
    scores = jax.nn.sigmoid(logits)

    r = jax.lax.broadcasted_iota(jnp.int32, (N_EXPERTS, N_EXPERTS), 0)
    c = jax.lax.broadcasted_iota(jnp.int32, (N_EXPERTS, N_EXPERTS), 1)
    lt = (r <= c).astype(jnp.bfloat16)
    iota_col = jax.lax.broadcasted_iota(
        jnp.int32, (N_EXPERTS, 1), 0).astype(jnp.bfloat16)

    def first_occurrence(mask_f32):
        m_bf = mask_f32.astype(jnp.bfloat16)
        cnt = jax.lax.dot_general(
            m_bf, lt, dimension_numbers=(((1,), (0,)), ((), ())),
            preferred_element_type=jnp.float32)
        return mask_f32 * (cnt == 1.0)

    m1 = jnp.max(scores, axis=1, keepdims=True)
    first1 = first_occurrence((scores >= m1).astype(jnp.float32))
    scores2 = scores - 2.0 * first1
    m2 = jnp.max(scores2, axis=1, keepdims=True)
    first2 = first_occurrence((scores2 >= m2).astype(jnp.float32))

    def index_of(first):
        f = jax.lax.dot_general(
            first.astype(jnp.bfloat16), iota_col,
            dimension_numbers=(((1,), (0,)), ((), ())),
            preferred_element_type=jnp.float32)
        return f.astype(jnp.int32)

    denom = jnp.clip(m1 + m2, 1e-12, None)
    w1 = m1 / denom
    w2 = m2 / denom
    out_w_ref[...] = jnp.concatenate([w1, w2], axis=1).astype(out_w_ref.dtype)
    out_i_ref[...] = jnp.concatenate(
        [index_of(first1), index_of(first2)], axis=1)


@jax.jit
def kernel(x, W):
    grid = (TOKENS // BLOCK_T,)
    out_w, out_i = pl.pallas_call(
        _router_kernel,
        grid=grid,
        in_specs=[
            pl.BlockSpec((BLOCK_T, DIM), lambda i: (i, 0)),
            pl.BlockSpec((N_EXPERTS, DIM), lambda i: (0, 0)),
        ],
        out_specs=[
            pl.BlockSpec((BLOCK_T, TOPK), lambda i: (i, 0)),
            pl.BlockSpec((BLOCK_T, TOPK), lambda i: (i, 0)),
        ],
        out_shape=[
            jax.ShapeDtypeStruct((TOKENS, TOPK), x.dtype),
            jax.ShapeDtypeStruct((TOKENS, TOPK), jnp.int32),
        ],
        compiler_params=pltpu.CompilerParams(
            dimension_semantics=("parallel",),
        ),
    )(x, W)
    return (out_w, out_i)


# speedup vs baseline: 1.1385x; 1.1385x over previous
import jax
import jax.numpy as jnp
from jax.experimental import pallas as pl
from jax.experimental.pallas import tpu as pltpu

DIM = 2048
N_EXPERTS = 64
TOPK = 2
TOKENS = 16384

BLOCK_T = 2048


def _router_kernel(x_ref, w_ref, out_w_ref, out_i_ref):
    x = x_ref[...]
    w = w_ref[...]
    logits = jax.lax.dot_general(
        x, w,
        dimension_numbers=(((1,), (1,)), ((), ())),
        preferred_element_type=jnp.float32,
    )
    scores = jax.nn.sigmoid(logits)

    iota = jax.lax.broadcasted_iota(jnp.int32, scores.shape, 1)
    m1 = jnp.max(scores, axis=1, keepdims=True)
    i1 = jnp.min(jnp.where(scores == m1, iota, N_EXPERTS), axis=1,
                 keepdims=True)
    masked = jnp.where(iota == i1, -1.0, scores)
    m2 = jnp.max(masked, axis=1, keepdims=True)
    i2 = jnp.min(jnp.where(masked == m2, iota, N_EXPERTS), axis=1,
                 keepdims=True)

    denom = jnp.clip(m1 + m2, 1e-12, None)
    w1 = m1 / denom
    w2 = m2 / denom
    out_w_ref[...] = jnp.concatenate([w1, w2], axis=1).astype(out_w_ref.dtype)
    out_i_ref[...] = jnp.concatenate([i1, i2], axis=1)


@jax.jit
def kernel(x, W):
    grid = (TOKENS // BLOCK_T,)
    out_w, out_i = pl.pallas_call(
        _router_kernel,
        grid=grid,
        in_specs=[
            pl.BlockSpec((BLOCK_T, DIM), lambda i: (i, 0)),
            pl.BlockSpec((N_EXPERTS, DIM), lambda i: (0, 0)),
        ],
        out_specs=[
            pl.BlockSpec((BLOCK_T, TOPK), lambda i: (i, 0)),
            pl.BlockSpec((BLOCK_T, TOPK), lambda i: (i, 0)),
        ],
        out_shape=[
            jax.ShapeDtypeStruct((TOKENS, TOPK), x.dtype),
            jax.ShapeDtypeStruct((TOKENS, TOPK), jnp.int32),
        ],
        compiler_params=pltpu.CompilerParams(
            dimension_semantics=("parallel",),
        ),
    )(x, W)
    return (out_w, out_i)
